# baseline (speedup 1.0000x reference)
import functools

import jax
import jax.numpy as jnp
from jax import lax
from jax.experimental import pallas as pl
from jax.experimental.pallas import tpu as pltpu
from jax.experimental.pallas import tpu_sc as plsc

B = 32; CITY = 1000; G = 16; E = 20000; TW = 24; FEAT = 8
X_EM = 32; LOC_EM = 8; DATE_EM = 8; EDGE_H = 16; GNN_H = 32; GNN_LAYER = 2
PRED = 6
N = B * CITY
F32 = jnp.float32

_EC = 10000


def _edge_body(z_hbm, row_hbm, col_hbm, ew_hbm, wcol_hbm, acc_hbm, deg_hbm,
               zloc, accv, degv, rowv, colv, eww, wcolv):
    c = lax.axis_index("c")
    s = lax.axis_index("s")
    b = c * 16 + s
    ZW = CITY * GNN_H
    pltpu.sync_copy(z_hbm.at[pl.ds(b * ZW, ZW)], zloc)
    pltpu.sync_copy(wcol_hbm, wcolv)
    wlo = wcolv[pl.ds(0, 16)]
    whi = wcolv[pl.ds(16, 16)]
    zero16 = jnp.zeros((16,), F32)
    ones16 = jnp.ones((16,), F32)

    def zacc(i, carry):
        accv[pl.ds(i * 16, 16)] = zero16
        return carry

    lax.fori_loop(0, ZW // 16, zacc, 0)

    def zdeg(i, carry):
        degv[pl.ds(i * 16, 16)] = zero16
        return carry

    lax.fori_loop(0, CITY, zdeg, 0)

    for ch in range(E // _EC):
        eo = b * E + ch * _EC
        pltpu.sync_copy(row_hbm.at[pl.ds(eo, _EC)], rowv)
        pltpu.sync_copy(col_hbm.at[pl.ds(eo, _EC)], colv)
        pltpu.sync_copy(ew_hbm.at[pl.ds(eo, _EC)], eww)

        def ebody(g, carry):
            base = g * 16
            rv = rowv[pl.ds(base, 16)] * GNN_H
            cv = colv[pl.ds(base, 16)] * GNN_H
            wv = eww[pl.ds(base, 16)]
            for l in range(16):
                rb = rv[l]
                cb = cv[l]
                we = wv[l]
                v0 = zloc[pl.ds(rb, 16)]
                v1 = zloc[pl.ds(rb + 16, 16)]
                m0 = jnp.maximum(v0 + we * wlo, 0.0)
                m1 = jnp.maximum(v1 + we * whi, 0.0)
                plsc.addupdate(accv.at[pl.ds(cb, 16)], m0)
                plsc.addupdate(accv.at[pl.ds(cb + 16, 16)], m1)
                plsc.addupdate(degv.at[pl.ds(cb // 2, 16)], ones16)
            return carry

        lax.fori_loop(0, _EC // 16, ebody, 0)

    pltpu.sync_copy(accv, acc_hbm.at[pl.ds(b * ZW, ZW)])
    pltpu.sync_copy(degv, deg_hbm.at[pl.ds(b * CITY * 16, CITY * 16)])


@jax.jit
def _edge_pass(z, row, col, ew, wcol):
    mesh = plsc.VectorSubcoreMesh(core_axis_name="c", subcore_axis_name="s")
    f = functools.partial(
        pl.kernel,
        mesh=mesh,
        out_type=[
            jax.ShapeDtypeStruct((N * GNN_H,), F32),
            jax.ShapeDtypeStruct((N * 16,), F32),
        ],
        scratch_types=[
            pltpu.VMEM((CITY * GNN_H,), F32),
            pltpu.VMEM((CITY * GNN_H,), F32),
            pltpu.VMEM((CITY * 16,), F32),
            pltpu.VMEM((_EC,), jnp.int32),
            pltpu.VMEM((_EC,), jnp.int32),
            pltpu.VMEM((_EC,), F32),
            pltpu.VMEM((32,), F32),
        ],
    )(_edge_body)
    return f(z, row, col, ew, wcol)


def _dot(a, bT):
    return lax.dot_general(a, bT, (((1,), (1,)), ((), ())),
                           preferred_element_type=F32)


def _dotn(a, b):
    return lax.dot_general(a, b, (((1,), (0,)), ((), ())),
                           preferred_element_type=F32)


def _dott(a, b):
    return lax.dot_general(a, b, (((0,), (0,)), ((), ())),
                           preferred_element_type=F32)


_LSTM_BLK = 4000


def _lstm_body(x_ref, W_ref, bias_ref, out_ref):
    W = W_ref[...]
    bias = bias_ref[...][None, :]
    h = jnp.zeros((_LSTM_BLK, X_EM), F32)
    c = jnp.zeros((_LSTM_BLK, X_EM), F32)
    for t in range(TW):
        xt = x_ref[:, t * FEAT:(t + 1) * FEAT]
        xh = jnp.concatenate([xt, h], axis=1)
        g = _dot(xh, W) + bias
        i = jax.nn.sigmoid(g[:, 0:X_EM])
        f = jax.nn.sigmoid(g[:, X_EM:2 * X_EM])
        gg = jnp.tanh(g[:, 2 * X_EM:3 * X_EM])
        o = jax.nn.sigmoid(g[:, 3 * X_EM:4 * X_EM])
        c = f * c + i * gg
        h = o * jnp.tanh(c)
    out_ref[...] = h


@jax.jit
def _lstm_tc(xr, W, bias):
    return pl.pallas_call(
        _lstm_body,
        grid=(N // _LSTM_BLK,),
        in_specs=[
            pl.BlockSpec((_LSTM_BLK, TW * FEAT), lambda i: (i, 0)),
            pl.BlockSpec((4 * X_EM, FEAT + X_EM), lambda i: (0, 0)),
            pl.BlockSpec((4 * X_EM,), lambda i: (0,)),
        ],
        out_specs=pl.BlockSpec((_LSTM_BLK, X_EM), lambda i: (i, 0)),
        out_shape=jax.ShapeDtypeStruct((N, X_EM), F32),
    )(xr, W, bias)


def _group_dense(gx, gew_full3, gew_diag, p):
    m1W, m1b, m2W, m2b = p
    D = gx.shape[-1]
    W1x, W1a = m1W[:, :D], m1W[:, D:]
    z = _dot(gx, W1x) + m1b[None, :]
    t_full = _dot(gew_full3.reshape(G * G, EDGE_H), W1a).reshape(G, G, GNN_H)
    t_diag = _dot(gew_diag, W1a)
    msum = jnp.zeros((G, GNN_H), F32)
    for i in range(G):
        msum = msum + jax.nn.relu(z[i:i + 1, :] + t_full[i])
    mean = (msum - jax.nn.relu(z + t_diag)) * (1.0 / (G - 1))
    out = jnp.concatenate([gx, mean], axis=1)
    return jax.nn.relu(_dot(out, m2W) + m2b[None, :])


def _encoder_body(h_ref, loc_ref, u_ref, w_ref, locW_ref, locb_ref,
                  u1_ref, u2_ref, u3_ref, eW_ref, eb_ref,
                  g1m1W_ref, g1m1b_ref, g1m2W_ref, g1m2b_ref,
                  g2m1W_ref, g2m1b_ref, g2m2W_ref, g2m2b_ref,
                  zW_ref, zb_ref,
                  gewf_ref, gewd_ref, x0_ref, z1_ref):
    wraw = w_ref[...]
    wm = jnp.max(wraw, axis=1, keepdims=True)
    wexp = jnp.exp(wraw - wm)
    wsoft = wexp / jnp.sum(wexp, axis=1, keepdims=True)

    locv = loc_ref[0]
    locW = locW_ref[...]
    loc_e = (locv[:, 0:1] * locW[:, 0][None, :]
             + locv[:, 1:2] * locW[:, 1][None, :] + locb_ref[...][None, :])
    h = h_ref[...]
    xloc = jnp.concatenate([h, loc_e], axis=1)
    gx = _dott(wsoft, xloc)

    u = u_ref[0]
    oh1 = (u[:, 0:1] == lax.broadcasted_iota(jnp.int32, (1, 12), 1)).astype(F32)
    oh2 = (u[:, 1:2] == lax.broadcasted_iota(jnp.int32, (1, 7), 1)).astype(F32)
    oh3 = (u[:, 2:3] == lax.broadcasted_iota(jnp.int32, (1, 24), 1)).astype(F32)
    u_em = jnp.concatenate([_dotn(oh1, u1_ref[...]), _dotn(oh2, u2_ref[...]),
                            _dotn(oh3, u3_ref[...])], axis=1)

    eW = eW_ref[...]
    d1 = X_EM + LOC_EM
    q1 = _dot(gx, eW[:, :d1])
    q2 = _dot(gx, eW[:, d1:2 * d1])
    ue = _dot(u_em, eW[:, 2 * d1:]) + eb_ref[...][None, :]
    gew_full = jax.nn.relu(q1[:, None, :] + q2[None, :, :] + ue[None, :, :])
    gew_diag = jax.nn.relu(q1 + q2 + ue)
    gewf_ref[...] = gew_full[None]
    gewd_ref[...] = gew_diag[None]

    p1 = (g1m1W_ref[...], g1m1b_ref[...], g1m2W_ref[...], g1m2b_ref[...])
    p2 = (g2m1W_ref[...], g2m1b_ref[...], g2m2W_ref[...], g2m2b_ref[...])
    gg = _group_dense(gx, gew_full, gew_diag, p1)
    gg = _group_dense(gg, gew_full, gew_diag, p2)

    x0 = jnp.concatenate([h, _dotn(wsoft, gg)], axis=1)
    x0_ref[...] = x0
    z1_ref[...] = _dot(x0, zW_ref[...]) + zb_ref[...][None, :]


@jax.jit
def _encoder_tc(h, loc3, u3d, wraw, locW, locb, u1, u2, u3, eW, eb,
                p1, p2, zW, zb):
    DX = X_EM + GNN_H
    full = lambda *shape: pl.BlockSpec(shape, lambda b: tuple(0 for _ in shape))
    outs = [
        jax.ShapeDtypeStruct((B, G, G, EDGE_H), F32),
        jax.ShapeDtypeStruct((B, G, EDGE_H), F32),
        jax.ShapeDtypeStruct((N, DX), F32),
        jax.ShapeDtypeStruct((N, GNN_H), F32),
    ]
    return pl.pallas_call(
        _encoder_body,
        grid=(B,),
        in_specs=[
            pl.BlockSpec((CITY, X_EM), lambda b: (b, 0)),
            pl.BlockSpec((1, CITY, 2), lambda b: (b, 0, 0)),
            pl.BlockSpec((1, 1, 3), lambda b: (b, 0, 0)),
            full(CITY, G),
            full(LOC_EM, 2), full(LOC_EM,),
            full(12, DATE_EM), full(7, DATE_EM), full(24, DATE_EM),
            full(EDGE_H, 2 * (X_EM + LOC_EM) + 3 * DATE_EM), full(EDGE_H,),
            full(GNN_H, X_EM + LOC_EM + EDGE_H), full(GNN_H,),
            full(GNN_H, X_EM + LOC_EM + GNN_H), full(GNN_H,),
            full(GNN_H, GNN_H + EDGE_H), full(GNN_H,),
            full(GNN_H, 2 * GNN_H), full(GNN_H,),
            full(GNN_H, DX), full(GNN_H,),
        ],
        out_specs=[
            pl.BlockSpec((1, G, G, EDGE_H), lambda b: (b, 0, 0, 0)),
            pl.BlockSpec((1, G, EDGE_H), lambda b: (b, 0, 0)),
            pl.BlockSpec((CITY, DX), lambda b: (b, 0)),
            pl.BlockSpec((CITY, GNN_H), lambda b: (b, 0)),
        ],
        out_shape=outs,
    )(h, loc3, u3d, wraw, locW, locb, u1, u2, u3, eW, eb,
      p1[0], p1[1], p1[2], p1[3], p2[0], p2[1], p2[2], p2[3], zW, zb)


_J_BLK = 8000


def _junction_body(x_ref, acc_ref, deg_ref, m2W_ref, m2b_ref, zW_ref, zb_ref,
                   xout_ref, zout_ref):
    D = x_ref.shape[1]
    inv = 1.0 / jnp.maximum(deg_ref[:, 0:1], 1.0)
    mean = acc_ref[...] * inv
    m2W = m2W_ref[...]
    xn = jax.nn.relu(_dot(x_ref[...], m2W[:, :D]) + _dot(mean, m2W[:, D:])
                     + m2b_ref[...][None, :])
    xout_ref[...] = xn
    zout_ref[...] = _dot(xn, zW_ref[...]) + zb_ref[...][None, :]


@jax.jit
def _junction_tc(x, acc, deg16, m2W, m2b, zW, zb):
    D = x.shape[1]
    full = lambda *shape: pl.BlockSpec(shape, lambda b: tuple(0 for _ in shape))
    outs = [
        jax.ShapeDtypeStruct((N, GNN_H), F32),
        jax.ShapeDtypeStruct((N, GNN_H), F32),
    ]
    return pl.pallas_call(
        _junction_body,
        grid=(N // _J_BLK,),
        in_specs=[
            pl.BlockSpec((_J_BLK, D), lambda i: (i, 0)),
            pl.BlockSpec((_J_BLK, GNN_H), lambda i: (i, 0)),
            pl.BlockSpec((_J_BLK, 16), lambda i: (i, 0)),
            full(*m2W.shape), full(*m2b.shape),
            full(*zW.shape), full(*zb.shape),
        ],
        out_specs=[
            pl.BlockSpec((_J_BLK, GNN_H), lambda i: (i, 0)),
            pl.BlockSpec((_J_BLK, GNN_H), lambda i: (i, 0)),
        ],
        out_shape=outs,
    )(x, acc, deg16, m2W, m2b, zW, zb)


def _decmid_body(x1_ref, acc_ref, deg_ref, g2m2W_ref, g2m2b_ref,
                 dW_ref, db_ref, w_ref,
                 d1m1W_ref, d1m1b_ref, d1m2W_ref, d1m2b_ref,
                 d2m1W_ref, d2m1b_ref, d2m2W_ref, d2m2b_ref,
                 gewf_ref, gewd_ref, zW_ref, zb_ref,
                 x0d_ref, z3_ref):
    inv = 1.0 / jnp.maximum(deg_ref[:, 0:1], 1.0)
    mean = acc_ref[...] * inv
    m2W = g2m2W_ref[...]
    x2 = jax.nn.relu(_dot(x1_ref[...], m2W[:, :GNN_H])
                     + _dot(mean, m2W[:, GNN_H:]) + g2m2b_ref[...][None, :])
    dx = _dot(x2, dW_ref[...]) + db_ref[...][None, :]
    wraw = w_ref[...]
    gx = _dott(wraw, dx)
    gewf = gewf_ref[0]
    gewd = gewd_ref[0]
    pd1 = (d1m1W_ref[...], d1m1b_ref[...], d1m2W_ref[...], d1m2b_ref[...])
    pd2 = (d2m1W_ref[...], d2m1b_ref[...], d2m2W_ref[...], d2m2b_ref[...])
    gg = _group_dense(gx, gewf, gewd, pd1)
    gg = _group_dense(gg, gewf, gewd, pd2)
    x0d = jnp.concatenate([dx, _dotn(wraw, gg)], axis=1)
    x0d_ref[...] = x0d
    z3_ref[...] = _dot(x0d, zW_ref[...]) + zb_ref[...][None, :]


@jax.jit
def _decmid_tc(x1, acc, deg16, g2m2W, g2m2b, dW, db, wraw,
               pd1, pd2, gewf, gewd, zW, zb):
    DX = X_EM + GNN_H
    full = lambda *shape: pl.BlockSpec(shape, lambda b: tuple(0 for _ in shape))
    outs = [
        jax.ShapeDtypeStruct((N, DX), F32),
        jax.ShapeDtypeStruct((N, GNN_H), F32),
    ]
    return pl.pallas_call(
        _decmid_body,
        grid=(B,),
        in_specs=[
            pl.BlockSpec((CITY, GNN_H), lambda b: (b, 0)),
            pl.BlockSpec((CITY, GNN_H), lambda b: (b, 0)),
            pl.BlockSpec((CITY, 16), lambda b: (b, 0)),
            full(*g2m2W.shape), full(*g2m2b.shape),
            full(*dW.shape), full(*db.shape),
            full(CITY, G),
            full(*pd1[0].shape), full(*pd1[1].shape),
            full(*pd1[2].shape), full(*pd1[3].shape),
            full(*pd2[0].shape), full(*pd2[1].shape),
            full(*pd2[2].shape), full(*pd2[3].shape),
            pl.BlockSpec((1, G, G, EDGE_H), lambda b: (b, 0, 0, 0)),
            pl.BlockSpec((1, G, EDGE_H), lambda b: (b, 0, 0)),
            full(*zW.shape), full(*zb.shape),
        ],
        out_specs=[
            pl.BlockSpec((CITY, DX), lambda b: (b, 0)),
            pl.BlockSpec((CITY, GNN_H), lambda b: (b, 0)),
        ],
        out_shape=outs,
    )(x1, acc, deg16, g2m2W, g2m2b, dW, db, wraw,
      pd1[0], pd1[1], pd1[2], pd1[3], pd2[0], pd2[1], pd2[2], pd2[3],
      gewf, gewd, zW, zb)


def _final_body(x3_ref, acc_ref, deg_ref, m2W_ref, m2b_ref,
                W1_ref, b1_ref, W2_ref, b2_ref, out_ref):
    inv = 1.0 / jnp.maximum(deg_ref[:, 0:1], 1.0)
    mean = acc_ref[...] * inv
    m2W = m2W_ref[...]
    x4 = jax.nn.relu(_dot(x3_ref[...], m2W[:, :GNN_H])
                     + _dot(mean, m2W[:, GNN_H:]) + m2b_ref[...][None, :])
    y = jax.nn.relu(_dot(x4, W1_ref[...]) + b1_ref[...][None, :])
    out_ref[...] = jax.nn.relu(_dot(y, W2_ref[...]) + b2_ref[...][None, :])


@jax.jit
def _final_tc(x3, acc, deg16, m2W, m2b, W1, b1, W2, b2):
    full = lambda *shape: pl.BlockSpec(shape, lambda b: tuple(0 for _ in shape))
    return pl.pallas_call(
        _final_body,
        grid=(N // _J_BLK,),
        in_specs=[
            pl.BlockSpec((_J_BLK, GNN_H), lambda i: (i, 0)),
            pl.BlockSpec((_J_BLK, GNN_H), lambda i: (i, 0)),
            pl.BlockSpec((_J_BLK, 16), lambda i: (i, 0)),
            full(*m2W.shape), full(*m2b.shape),
            full(*W1.shape), full(*b1.shape),
            full(*W2.shape), full(*b2.shape),
        ],
        out_specs=pl.BlockSpec((_J_BLK, PRED), lambda i: (i, 0)),
        out_shape=jax.ShapeDtypeStruct((N, PRED), F32),
    )(x3, acc, deg16, m2W, m2b, W1, b1, W2, b2)


def kernel(x, u, edge_index, edge_w, loc, params):
    Wih, Whh, bih, bhh = params['lstm']
    W_l = jnp.concatenate([Wih, Whh], axis=1)
    h = _lstm_tc(x.reshape(N, TW * FEAT), W_l, bih + bhh)

    row = edge_index[:, 0].reshape(-1)
    col = edge_index[:, 1].reshape(-1)
    ewf = edge_w.reshape(-1)

    g1 = params['global_gnn'][0]
    g2 = params['global_gnn'][1]
    dg1 = params['dec_global_gnn'][0]
    dg2 = params['dec_global_gnn'][1]
    DX = X_EM + GNN_H

    gewf, gewd, x0, z1 = _encoder_tc(
        h, loc.reshape(B, CITY, 2), u.reshape(B, 1, 3), params['w'],
        params['loc'][0], params['loc'][1],
        params['u1'], params['u2'], params['u3'],
        params['edge_inf'][0], params['edge_inf'][1],
        params['group_gnn'][0], params['group_gnn'][1],
        g1[0][:, :DX], g1[1])

    acc1, degx = _edge_pass(z1.reshape(-1), row, col, ewf, g1[0][:, DX])
    deg16 = degx.reshape(N, 16)

    x1, z2 = _junction_tc(x0, acc1.reshape(N, GNN_H), deg16,
                          g1[2], g1[3], g2[0][:, :GNN_H], g2[1])
    acc2, _ = _edge_pass(z2.reshape(-1), row, col, ewf, g2[0][:, GNN_H])

    x0d, z3 = _decmid_tc(x1, acc2.reshape(N, GNN_H), deg16, g2[2], g2[3],
                         params['dec_x_embed'][0], params['dec_x_embed'][1],
                         params['w'], params['dec_group_gnn'][0],
                         params['dec_group_gnn'][1], gewf, gewd,
                         dg1[0][:, :DX], dg1[1])
    acc3, _ = _edge_pass(z3.reshape(-1), row, col, ewf, dg1[0][:, DX])

    x3, z4 = _junction_tc(x0d, acc3.reshape(N, GNN_H), deg16,
                          dg1[2], dg1[3], dg2[0][:, :GNN_H], dg2[1])
    acc4, _ = _edge_pass(z4.reshape(-1), row, col, ewf, dg2[0][:, GNN_H])

    W1, b1, W2, b2 = params['pred']
    res = _final_tc(x3, acc4.reshape(N, GNN_H), deg16,
                    dg2[2], dg2[3], W1, b1, W2, b2)
    return res.reshape(B, CITY, PRED)
